# baseline (speedup 1.0000x reference)
import functools

import jax
import jax.numpy as jnp
from jax import lax
from jax.experimental import pallas as pl
from jax.experimental.pallas import tpu as pltpu
from jax.experimental.pallas import tpu_sc as plsc


def _sc_broadcast_copy(table, n_batch, seq_len):
    max_len, d_model = table.shape
    info = plsc.get_sparse_core_info()
    nc, ns = info.num_cores, info.num_subcores
    nw = nc * ns
    rows_per_w = seq_len // nw
    chunk = 32
    n_chunks = rows_per_w // chunk

    mesh = plsc.VectorSubcoreMesh(core_axis_name="c", subcore_axis_name="s")

    @functools.partial(
        pl.kernel,
        mesh=mesh,
        out_type=jax.ShapeDtypeStruct((n_batch, seq_len, d_model), table.dtype),
        scratch_types=[
            pltpu.VMEM((2, chunk, d_model), jnp.float32),
            pltpu.SemaphoreType.DMA,
            pltpu.SemaphoreType.DMA,
        ],
    )
    def sc_copy(table_hbm, out_hbm, bufs, insem, outsem):
        wid = lax.axis_index("s") * nc + lax.axis_index("c")
        base = wid * rows_per_w

        def cp_in(i):
            start = base + i * chunk
            return pltpu.async_copy(
                table_hbm.at[pl.ds(start, chunk)], bufs.at[i % 2], insem
            )

        def cp_out(i, b):
            start = base + i * chunk
            return pltpu.async_copy(
                bufs.at[i % 2], out_hbm.at[b, pl.ds(start, chunk)], outsem
            )

        h_in = [None] * n_chunks
        h_out = [None] * n_chunks
        h_in[0] = cp_in(0)
        for i in range(n_chunks):
            if i + 1 < n_chunks:
                if i >= 1:
                    for h in h_out[i - 1]:
                        h.wait()
                h_in[i + 1] = cp_in(i + 1)
            h_in[i].wait()
            h_out[i] = [cp_out(i, b) for b in range(n_batch)]
        for i in (n_chunks - 2, n_chunks - 1):
            for h in h_out[i]:
                h.wait()

    return sc_copy(table)


def _tc_broadcast_copy(table, n_batch, seq_len):
    max_len, d_model = table.shape
    blk = 256

    def body(t_ref, o_ref):
        o_ref[...] = jnp.broadcast_to(t_ref[...][None, :, :], o_ref.shape)

    return pl.pallas_call(
        body,
        grid=(seq_len // blk,),
        in_specs=[pl.BlockSpec((blk, d_model), lambda i: (i, 0))],
        out_specs=pl.BlockSpec((n_batch, blk, d_model), lambda i: (0, i, 0)),
        out_shape=jax.ShapeDtypeStruct((n_batch, seq_len, d_model), table.dtype),
    )(table)


def kernel(input_ids, table):
    batch_size, seq_len = input_ids.shape
    sc_batches = 1
    tc_batches = batch_size - sc_batches
    sc_out = _sc_broadcast_copy(table, sc_batches, seq_len)
    tc_out = _tc_broadcast_copy(table, tc_batches, seq_len)
    return jnp.concatenate([tc_out, sc_out], axis=0)


# speedup vs baseline: 2.2555x; 2.2555x over previous
import functools

import jax
import jax.numpy as jnp
from jax import lax
from jax.experimental import pallas as pl
from jax.experimental.pallas import tpu as pltpu
from jax.experimental.pallas import tpu_sc as plsc


def kernel(input_ids, table):
    batch_size, seq_len = input_ids.shape
    max_len, d_model = table.shape

    info = plsc.get_sparse_core_info()
    nc, ns = info.num_cores, info.num_subcores
    nw = nc * ns
    rows_per_w = seq_len // nw
    chunk = 32
    n_chunks = rows_per_w // chunk

    mesh = plsc.VectorSubcoreMesh(core_axis_name="c", subcore_axis_name="s")

    @functools.partial(
        pl.kernel,
        mesh=mesh,
        out_type=jax.ShapeDtypeStruct((batch_size, seq_len, d_model), table.dtype),
        scratch_types=[
            pltpu.VMEM((2, chunk, d_model), jnp.float32),
            pltpu.SemaphoreType.DMA,
            pltpu.SemaphoreType.DMA,
        ],
    )
    def sc_copy(table_hbm, out_hbm, bufs, insem, outsem):
        wid = lax.axis_index("s") * nc + lax.axis_index("c")
        base = wid * rows_per_w

        def cp_in(i):
            start = base + i * chunk
            return pltpu.async_copy(
                table_hbm.at[pl.ds(start, chunk)], bufs.at[i % 2], insem
            )

        def cp_out(i, b):
            start = base + i * chunk
            return pltpu.async_copy(
                bufs.at[i % 2], out_hbm.at[b, pl.ds(start, chunk)], outsem
            )

        h_in = [None] * n_chunks
        h_out = [None] * n_chunks
        h_in[0] = cp_in(0)
        for i in range(n_chunks):
            if i + 1 < n_chunks:
                if i >= 1:
                    for h in h_out[i - 1]:
                        h.wait()
                h_in[i + 1] = cp_in(i + 1)
            h_in[i].wait()
            h_out[i] = [cp_out(i, b) for b in range(batch_size)]
        for i in (n_chunks - 2, n_chunks - 1):
            for h in h_out[i]:
                h.wait()

    return sc_copy(table)


# speedup vs baseline: 2.2701x; 1.0065x over previous
import functools

import jax
import jax.numpy as jnp
from jax import lax
from jax.experimental import pallas as pl
from jax.experimental.pallas import tpu as pltpu
from jax.experimental.pallas import tpu_sc as plsc


def kernel(input_ids, table):
    batch_size, seq_len = input_ids.shape
    max_len, d_model = table.shape

    info = plsc.get_sparse_core_info()
    nc, ns = info.num_cores, info.num_subcores
    nw = nc * ns
    rows_per_w = seq_len // nw
    chunk = 32
    n_chunks = rows_per_w // chunk
    nbuf = 3

    mesh = plsc.VectorSubcoreMesh(core_axis_name="c", subcore_axis_name="s")

    @functools.partial(
        pl.kernel,
        mesh=mesh,
        out_type=jax.ShapeDtypeStruct((batch_size, seq_len, d_model), table.dtype),
        scratch_types=[
            pltpu.VMEM((nbuf, chunk, d_model), jnp.float32),
            pltpu.SemaphoreType.DMA,
            pltpu.SemaphoreType.DMA,
        ],
    )
    def sc_copy(table_hbm, out_hbm, bufs, insem, outsem):
        wid = lax.axis_index("s") * nc + lax.axis_index("c")
        base = wid * rows_per_w

        def cp_in(i):
            start = base + i * chunk
            return pltpu.async_copy(
                table_hbm.at[pl.ds(start, chunk)], bufs.at[i % nbuf], insem
            )

        def cp_out(i, b):
            start = base + i * chunk
            return pltpu.async_copy(
                bufs.at[i % nbuf], out_hbm.at[b, pl.ds(start, chunk)], outsem
            )

        h_in = [None] * n_chunks
        h_out = [None] * n_chunks
        h_in[0] = cp_in(0)
        for i in range(n_chunks):
            if i + 1 < n_chunks:
                if i + 1 - nbuf >= 0:
                    for h in h_out[i + 1 - nbuf]:
                        h.wait()
                h_in[i + 1] = cp_in(i + 1)
            h_in[i].wait()
            h_out[i] = [cp_out(i, b) for b in range(batch_size)]
        for i in range(max(0, n_chunks - nbuf), n_chunks):
            for h in h_out[i]:
                h.wait()

    return sc_copy(table)


# speedup vs baseline: 2.3229x; 1.0233x over previous
import functools

import jax
import jax.numpy as jnp
from jax import lax
from jax.experimental import pallas as pl
from jax.experimental.pallas import tpu as pltpu
from jax.experimental.pallas import tpu_sc as plsc


def kernel(input_ids, table):
    batch_size, seq_len = input_ids.shape
    max_len, d_model = table.shape

    info = plsc.get_sparse_core_info()
    nc, ns = info.num_cores, info.num_subcores
    nw = nc * ns
    rows_per_w = seq_len // nw
    chunk = 56
    nbuf = 2
    bounds = list(range(0, rows_per_w, chunk)) + [rows_per_w]
    sizes = [bounds[j + 1] - bounds[j] for j in range(len(bounds) - 1)]
    n_chunks = len(sizes)

    mesh = plsc.VectorSubcoreMesh(core_axis_name="c", subcore_axis_name="s")

    @functools.partial(
        pl.kernel,
        mesh=mesh,
        out_type=jax.ShapeDtypeStruct((batch_size, seq_len, d_model), table.dtype),
        scratch_types=[
            pltpu.VMEM((nbuf, chunk, d_model), jnp.float32),
            pltpu.SemaphoreType.DMA,
            pltpu.SemaphoreType.DMA,
        ],
    )
    def sc_copy(table_hbm, out_hbm, bufs, insem, outsem):
        wid = lax.axis_index("s") * nc + lax.axis_index("c")
        base = wid * rows_per_w

        def cp_in(i):
            start = base + bounds[i]
            return pltpu.async_copy(
                table_hbm.at[pl.ds(start, sizes[i])],
                bufs.at[i % nbuf, pl.ds(0, sizes[i])],
                insem,
            )

        def cp_out(i, b):
            start = base + bounds[i]
            return pltpu.async_copy(
                bufs.at[i % nbuf, pl.ds(0, sizes[i])],
                out_hbm.at[b, pl.ds(start, sizes[i])],
                outsem,
            )

        h_in = [None] * n_chunks
        h_out = [None] * n_chunks
        h_in[0] = cp_in(0)
        for i in range(n_chunks):
            if i + 1 < n_chunks:
                if i + 1 - nbuf >= 0:
                    for h in h_out[i + 1 - nbuf]:
                        h.wait()
                h_in[i + 1] = cp_in(i + 1)
            h_in[i].wait()
            h_out[i] = [cp_out(i, b) for b in range(batch_size)]
        for i in range(max(0, n_chunks - nbuf), n_chunks):
            for h in h_out[i]:
                h.wait()

    return sc_copy(table)
